# baseline (speedup 1.0000x reference)
import functools

import jax
import jax.numpy as jnp
from jax import lax
from jax.experimental import pallas as pl
from jax.experimental.pallas import tpu as pltpu
from jax.experimental.pallas import tpu_sc as plsc

_N_E = 8192
_E_DIM = 256
_B = 8192
_BETA = 0.25

_TBS = 1024
_CBS = 1024
_SLAB = 128
_NI = _B // _TBS
_NJ = _N_E // _CBS
_NS = _CBS // _SLAB
_NR = _SLAB // 8


def _lex_merge(va, ia, vb, ib):
    take = jnp.logical_or(vb < va, jnp.logical_and(vb == va, ib < ia))
    return jnp.where(take, vb, va), jnp.where(take, ib, ia)


def _dist_argmin_body(z_ref, w_ref, zsq_ref, wsq_ref,
                      idx_ref, loss_ref, rmin_ref, ridx_ref):
    i = pl.program_id(0)
    j = pl.program_id(1)
    zbt2 = z_ref[i] + z_ref[i]
    zsq_b = jnp.broadcast_to(zsq_ref[:, pl.ds(i * _TBS, _TBS)], (8, _TBS))

    pmin = None
    pgid = None
    for s in range(_NS):
        c0 = j * _CBS + s * _SLAB
        wb = w_ref[pl.ds(c0, _SLAB), :]
        mm2 = lax.dot_general(wb, zbt2, (((1,), (0,)), ((), ())),
                              preferred_element_type=jnp.float32)
        for r in range(_NR):
            wsqv = wsq_ref[pl.ds(c0 + 8 * r, 8), :]
            dv = (wsqv + zsq_b) - mm2[8 * r:8 * r + 8]
            gid = (c0 + 8 * r) // 8
            if pmin is None:
                pmin, pgid = dv, jnp.full((8, _TBS), gid, jnp.int32)
            else:
                take = dv < pmin
                pmin = jnp.minimum(pmin, dv)
                pgid = jnp.where(take, jnp.int32(gid), pgid)

    first = j == 0
    inf8 = jnp.full((8, _TBS), jnp.inf, jnp.float32)
    prev_min = jnp.where(first, inf8, rmin_ref[...])
    prev_gid = ridx_ref[...]
    take = pmin < prev_min
    new_min = jnp.minimum(pmin, prev_min)
    new_gid = jnp.where(take, pgid, prev_gid)
    rmin_ref[...] = new_min
    ridx_ref[...] = new_gid

    @pl.when(j == _NJ - 1)
    def _():
        sub = lax.broadcasted_iota(jnp.int32, (8, _TBS), 0)
        new_idx = new_gid * 8 + sub
        v0, i0 = _lex_merge(new_min[0:4], new_idx[0:4],
                            new_min[4:8], new_idx[4:8])
        v1, i1 = _lex_merge(v0[0:2], i0[0:2], v0[2:4], i0[2:4])
        v2, i2 = _lex_merge(v1[0:1], i1[0:1], v1[1:2], i1[1:2])
        idx_ref[...] = i2
        prev = jnp.where(i == 0, jnp.zeros((1, 1), jnp.float32), loss_ref[...])
        loss_ref[...] = prev + jnp.sum(v2, keepdims=True)


def _dist_argmin(z_bct, W, zsq_row, wsq_col):
    return pl.pallas_call(
        _dist_argmin_body,
        grid=(_NI, _NJ),
        in_specs=[
            pl.BlockSpec((_NI, _E_DIM, _TBS), lambda i, j: (0, 0, 0)),
            pl.BlockSpec((_N_E, _E_DIM), lambda i, j: (0, 0)),
            pl.BlockSpec((1, _B), lambda i, j: (0, 0)),
            pl.BlockSpec((_N_E, 1), lambda i, j: (0, 0)),
        ],
        out_specs=[
            pl.BlockSpec((1, _TBS), lambda i, j: (0, i)),
            pl.BlockSpec((1, 1), lambda i, j: (0, 0)),
        ],
        out_shape=[
            jax.ShapeDtypeStruct((1, _B), jnp.int32),
            jax.ShapeDtypeStruct((1, 1), jnp.float32),
        ],
        scratch_shapes=[
            pltpu.VMEM((8, _TBS), jnp.float32),
            pltpu.VMEM((8, _TBS), jnp.int32),
        ],
    )(z_bct, W, zsq_row, wsq_col)


_NW = 32
_BPW = _B // _NW
_CH = 128


def _gather(W, idx):
    mesh = plsc.VectorSubcoreMesh(core_axis_name="c", subcore_axis_name="s")

    @functools.partial(
        pl.kernel,
        out_type=jax.ShapeDtypeStruct((_B, _E_DIM), jnp.float32),
        mesh=mesh,
        scratch_types=[
            pltpu.VMEM((_CH,), jnp.int32),
            pltpu.VMEM((_CH, _E_DIM), jnp.float32),
            pltpu.SemaphoreType.DMA,
        ],
    )
    def k(w_hbm, idx_hbm, out_hbm, idx_v, rows_v, sem):
        wid = lax.axis_index("s") * 2 + lax.axis_index("c")
        for ch in range(_BPW // _CH):
            base = wid * _BPW + ch * _CH
            pltpu.sync_copy(idx_hbm.at[pl.ds(base, _CH)], idx_v)
            pltpu.async_copy(w_hbm.at[idx_v], rows_v, sem).wait()
            pltpu.sync_copy(rows_v, out_hbm.at[pl.ds(base, _CH)])

    return k(W, idx)


def _st_body(z_ref, zq_ref, out_ref):
    zq_t = jnp.transpose(zq_ref[...], (1, 0))
    zb = z_ref[0]
    out_ref[0] = zb + (zq_t - zb)


def _st_transpose(z_bct, zq_g):
    return pl.pallas_call(
        _st_body,
        grid=(_NI,),
        in_specs=[
            pl.BlockSpec((1, _E_DIM, _TBS), lambda b: (b, 0, 0)),
            pl.BlockSpec((_TBS, _E_DIM), lambda b: (b, 0)),
        ],
        out_specs=pl.BlockSpec((1, _E_DIM, _TBS), lambda b: (b, 0, 0)),
        out_shape=jax.ShapeDtypeStruct((_NI, _E_DIM, _TBS), jnp.float32),
    )(z_bct, zq_g)


def kernel(z, W):
    z_flat = jnp.transpose(z, (0, 2, 3, 1)).reshape(-1, _E_DIM)
    zsq = jnp.sum(z_flat ** 2, axis=1, keepdims=True)
    wsq = jnp.sum(W ** 2, axis=1)
    z_bct = z.reshape(_NI, _E_DIM, _TBS)
    idx2d, loss_sum = _dist_argmin(
        z_bct, W, zsq.reshape(1, _B), wsq.reshape(_N_E, 1))
    idx = idx2d.reshape(_B)
    zq_g = _gather(W, idx)
    m = loss_sum[0, 0] / jnp.float32(_B * _E_DIM)
    loss = m + _BETA * m
    zq_bct = _st_transpose(z_bct, zq_g)
    z_q = zq_bct.reshape(z.shape)
    return z_q, loss, idx


# speedup vs baseline: 1.0571x; 1.0571x over previous
import functools

import jax
import jax.numpy as jnp
from jax import lax
from jax.experimental import pallas as pl
from jax.experimental.pallas import tpu as pltpu
from jax.experimental.pallas import tpu_sc as plsc

_N_E = 8192
_E_DIM = 256
_B = 8192
_BETA = 0.25

_TBS = 1024
_CBS = 1024
_SLAB = 128
_NI = _B // _TBS
_NJ = _N_E // _CBS
_NS = _CBS // _SLAB
_NR = _SLAB // 8


def _lex_merge(va, ia, vb, ib):
    take = jnp.logical_or(vb < va, jnp.logical_and(vb == va, ib < ia))
    return jnp.where(take, vb, va), jnp.where(take, ib, ia)


def _dist_argmin_body(z_ref, w_ref, zsq_ref, wsq_ref,
                      idx_ref, loss_ref, rmin_ref, ridx_ref):
    i = pl.program_id(0)
    j = pl.program_id(1)
    zbt2 = z_ref[i] + z_ref[i]
    zsq_b = jnp.broadcast_to(zsq_ref[:, pl.ds(i * _TBS, _TBS)], (8, _TBS))

    pmin = None
    pgid = None
    for s in range(_NS):
        c0 = j * _CBS + s * _SLAB
        wb = w_ref[pl.ds(c0, _SLAB), :]
        mm2 = lax.dot_general(wb, zbt2, (((1,), (0,)), ((), ())),
                              preferred_element_type=jnp.float32)
        for r in range(_NR):
            wsqv = wsq_ref[pl.ds(c0 + 8 * r, 8), :]
            dv = (wsqv + zsq_b) - mm2[8 * r:8 * r + 8]
            gid = (c0 + 8 * r) // 8
            if pmin is None:
                pmin, pgid = dv, jnp.full((8, _TBS), gid, jnp.int32)
            else:
                take = dv < pmin
                pmin = jnp.minimum(pmin, dv)
                pgid = jnp.where(take, jnp.int32(gid), pgid)

    first = j == 0
    inf8 = jnp.full((8, _TBS), jnp.inf, jnp.float32)
    prev_min = jnp.where(first, inf8, rmin_ref[...])
    prev_gid = ridx_ref[...]
    take = pmin < prev_min
    new_min = jnp.minimum(pmin, prev_min)
    new_gid = jnp.where(take, pgid, prev_gid)
    rmin_ref[...] = new_min
    ridx_ref[...] = new_gid

    @pl.when(j == _NJ - 1)
    def _():
        sub = lax.broadcasted_iota(jnp.int32, (8, _TBS), 0)
        new_idx = new_gid * 8 + sub
        v0, i0 = _lex_merge(new_min[0:4], new_idx[0:4],
                            new_min[4:8], new_idx[4:8])
        v1, i1 = _lex_merge(v0[0:2], i0[0:2], v0[2:4], i0[2:4])
        v2, i2 = _lex_merge(v1[0:1], i1[0:1], v1[1:2], i1[1:2])
        idx_ref[...] = i2
        prev = jnp.where(i == 0, jnp.zeros((1, 1), jnp.float32), loss_ref[...])
        loss_ref[...] = prev + jnp.sum(v2, keepdims=True)


def _dist_argmin(z_bct, W, zsq_row, wsq_col):
    return pl.pallas_call(
        _dist_argmin_body,
        grid=(_NI, _NJ),
        in_specs=[
            pl.BlockSpec((_NI, _E_DIM, _TBS), lambda i, j: (0, 0, 0)),
            pl.BlockSpec((_N_E, _E_DIM), lambda i, j: (0, 0)),
            pl.BlockSpec((1, _B), lambda i, j: (0, 0)),
            pl.BlockSpec((_N_E, 1), lambda i, j: (0, 0)),
        ],
        out_specs=[
            pl.BlockSpec((1, _TBS), lambda i, j: (0, i)),
            pl.BlockSpec((1, 1), lambda i, j: (0, 0)),
        ],
        out_shape=[
            jax.ShapeDtypeStruct((1, _B), jnp.int32),
            jax.ShapeDtypeStruct((1, 1), jnp.float32),
        ],
        scratch_shapes=[
            pltpu.VMEM((8, _TBS), jnp.float32),
            pltpu.VMEM((8, _TBS), jnp.int32),
        ],
    )(z_bct, W, zsq_row, wsq_col)


_NW = 32
_BPW = _B // _NW
_CH = 128


def _gather_st(W, idx, z_flat):
    nt = idx.shape[0]
    mesh = plsc.VectorSubcoreMesh(core_axis_name="c", subcore_axis_name="s")
    bpw = nt // _NW

    @functools.partial(
        pl.kernel,
        out_type=jax.ShapeDtypeStruct((nt, _E_DIM), jnp.float32),
        mesh=mesh,
        scratch_types=[
            pltpu.VMEM((_CH,), jnp.int32),
            pltpu.VMEM((_CH, _E_DIM), jnp.float32),
            pltpu.VMEM((_CH, _E_DIM), jnp.float32),
            pltpu.SemaphoreType.DMA,
        ],
    )
    def k(w_hbm, idx_hbm, z_hbm, out_hbm, idx_v, rows_v, z_v, sem):
        wid = lax.axis_index("s") * 2 + lax.axis_index("c")
        for ch in range(bpw // _CH):
            base = wid * bpw + ch * _CH
            pltpu.sync_copy(idx_hbm.at[pl.ds(base, _CH)], idx_v)
            pltpu.sync_copy(z_hbm.at[pl.ds(base, _CH)], z_v)
            pltpu.async_copy(w_hbm.at[idx_v], rows_v, sem).wait()

            def body(r, carry):
                for c in range(_E_DIM // 16):
                    sl = pl.ds(c * 16, 16)
                    zv = z_v[r, sl]
                    wv = rows_v[r, sl]
                    rows_v[r, sl] = zv + (wv - zv)
                return carry

            lax.fori_loop(0, _CH, body, 0)
            pltpu.sync_copy(rows_v, out_hbm.at[pl.ds(base, _CH)])

    return k(W, idx, z_flat)


def kernel(z, W):
    z_flat = jnp.transpose(z, (0, 2, 3, 1)).reshape(-1, _E_DIM)
    zsq = jnp.sum(z_flat ** 2, axis=1, keepdims=True)
    wsq = jnp.sum(W ** 2, axis=1)
    z_bct = z.reshape(_NI, _E_DIM, _TBS)
    idx2d, loss_sum = _dist_argmin(
        z_bct, W, zsq.reshape(1, _B), wsq.reshape(_N_E, 1))
    idx = idx2d.reshape(_B)
    zq_st = _gather_st(W, idx, z_flat)
    m = loss_sum[0, 0] / jnp.float32(_B * _E_DIM)
    loss = m + _BETA * m
    z_q = zq_st.reshape(8, 32, 32, _E_DIM).transpose(0, 3, 1, 2)
    return z_q, loss, idx


# speedup vs baseline: 1.1645x; 1.1016x over previous
import functools

import jax
import jax.numpy as jnp
from jax import lax
from jax.experimental import pallas as pl
from jax.experimental.pallas import tpu as pltpu
from jax.experimental.pallas import tpu_sc as plsc

_N_E = 8192
_E_DIM = 256
_B = 8192
_BETA = 0.25

_TBS = 1024
_CBS = 2048
_SLAB = 256
_NI = _B // _TBS
_NJ = _N_E // _CBS
_NS = _CBS // _SLAB
_NR = _SLAB // 8


def _lex_merge(va, ia, vb, ib):
    take = jnp.logical_or(vb < va, jnp.logical_and(vb == va, ib < ia))
    return jnp.where(take, vb, va), jnp.where(take, ib, ia)


def _dist_argmin_body(z_ref, w_ref, zsq_ref, wsq_ref,
                      idx_ref, loss_ref, rmin_ref, ridx_ref):
    i = pl.program_id(0)
    j = pl.program_id(1)
    zbt2 = z_ref[i] + z_ref[i]
    zsq_b = jnp.broadcast_to(zsq_ref[:, pl.ds(i * _TBS, _TBS)], (8, _TBS))

    pmin = None
    pgid = None
    for s in range(_NS):
        c0 = j * _CBS + s * _SLAB
        wb = w_ref[pl.ds(c0, _SLAB), :]
        mm2 = lax.dot_general(wb, zbt2, (((1,), (0,)), ((), ())),
                              preferred_element_type=jnp.float32)
        for r in range(_NR):
            wsqv = wsq_ref[pl.ds(c0 + 8 * r, 8), :]
            dv = (wsqv + zsq_b) - mm2[8 * r:8 * r + 8]
            gid = (c0 + 8 * r) // 8
            if pmin is None:
                pmin, pgid = dv, jnp.full((8, _TBS), gid, jnp.int32)
            else:
                take = dv < pmin
                pmin = jnp.minimum(pmin, dv)
                pgid = jnp.where(take, jnp.int32(gid), pgid)

    first = j == 0
    inf8 = jnp.full((8, _TBS), jnp.inf, jnp.float32)
    prev_min = jnp.where(first, inf8, rmin_ref[...])
    prev_gid = ridx_ref[...]
    take = pmin < prev_min
    new_min = jnp.minimum(pmin, prev_min)
    new_gid = jnp.where(take, pgid, prev_gid)
    rmin_ref[...] = new_min
    ridx_ref[...] = new_gid

    @pl.when(j == _NJ - 1)
    def _():
        sub = lax.broadcasted_iota(jnp.int32, (8, _TBS), 0)
        new_idx = new_gid * 8 + sub
        v0, i0 = _lex_merge(new_min[0:4], new_idx[0:4],
                            new_min[4:8], new_idx[4:8])
        v1, i1 = _lex_merge(v0[0:2], i0[0:2], v0[2:4], i0[2:4])
        v2, i2 = _lex_merge(v1[0:1], i1[0:1], v1[1:2], i1[1:2])
        idx_ref[...] = i2
        prev = jnp.where(i == 0, jnp.zeros((1, 1), jnp.float32), loss_ref[...])
        loss_ref[...] = prev + jnp.sum(v2, keepdims=True)


def _dist_argmin(z_bct, W, zsq_row, wsq_col):
    return pl.pallas_call(
        _dist_argmin_body,
        grid=(_NI, _NJ),
        in_specs=[
            pl.BlockSpec((_NI, _E_DIM, _TBS), lambda i, j: (0, 0, 0)),
            pl.BlockSpec((_N_E, _E_DIM), lambda i, j: (0, 0)),
            pl.BlockSpec((1, _B), lambda i, j: (0, 0)),
            pl.BlockSpec((_N_E, 1), lambda i, j: (0, 0)),
        ],
        out_specs=[
            pl.BlockSpec((1, _TBS), lambda i, j: (0, i)),
            pl.BlockSpec((1, 1), lambda i, j: (0, 0)),
        ],
        out_shape=[
            jax.ShapeDtypeStruct((1, _B), jnp.int32),
            jax.ShapeDtypeStruct((1, 1), jnp.float32),
        ],
        scratch_shapes=[
            pltpu.VMEM((8, _TBS), jnp.float32),
            pltpu.VMEM((8, _TBS), jnp.int32),
        ],
    )(z_bct, W, zsq_row, wsq_col)


_NW = 32
_BPW = _B // _NW
_CH = 128


def _gather_st(W, idx, z_flat):
    nt = idx.shape[0]
    mesh = plsc.VectorSubcoreMesh(core_axis_name="c", subcore_axis_name="s")
    bpw = nt // _NW

    @functools.partial(
        pl.kernel,
        out_type=jax.ShapeDtypeStruct((nt, _E_DIM), jnp.float32),
        mesh=mesh,
        scratch_types=[
            pltpu.VMEM((_CH,), jnp.int32),
            pltpu.VMEM((_CH, _E_DIM), jnp.float32),
            pltpu.VMEM((_CH, _E_DIM), jnp.float32),
            pltpu.SemaphoreType.DMA,
        ],
    )
    def k(w_hbm, idx_hbm, z_hbm, out_hbm, idx_v, rows_v, z_v, sem):
        wid = lax.axis_index("s") * 2 + lax.axis_index("c")
        for ch in range(bpw // _CH):
            base = wid * bpw + ch * _CH
            pltpu.sync_copy(idx_hbm.at[pl.ds(base, _CH)], idx_v)
            pltpu.sync_copy(z_hbm.at[pl.ds(base, _CH)], z_v)
            pltpu.async_copy(w_hbm.at[idx_v], rows_v, sem).wait()

            def body(r, carry):
                for c in range(_E_DIM // 16):
                    sl = pl.ds(c * 16, 16)
                    zv = z_v[r, sl]
                    wv = rows_v[r, sl]
                    rows_v[r, sl] = zv + (wv - zv)
                return carry

            lax.fori_loop(0, _CH, body, 0)
            pltpu.sync_copy(rows_v, out_hbm.at[pl.ds(base, _CH)])

    return k(W, idx, z_flat)


def kernel(z, W):
    z_flat = jnp.transpose(z, (0, 2, 3, 1)).reshape(-1, _E_DIM)
    zsq = jnp.sum(z_flat ** 2, axis=1, keepdims=True)
    wsq = jnp.sum(W ** 2, axis=1)
    z_bct = z.reshape(_NI, _E_DIM, _TBS)
    idx2d, loss_sum = _dist_argmin(
        z_bct, W, zsq.reshape(1, _B), wsq.reshape(_N_E, 1))
    idx = idx2d.reshape(_B)
    zq_st = _gather_st(W, idx, z_flat)
    m = loss_sum[0, 0] / jnp.float32(_B * _E_DIM)
    loss = m + _BETA * m
    z_q = zq_st.reshape(8, 32, 32, _E_DIM).transpose(0, 3, 1, 2)
    return z_q, loss, idx
